# baseline (speedup 1.0000x reference)
import functools

import jax
import jax.numpy as jnp
from jax import lax
from jax.experimental import pallas as pl
from jax.experimental.pallas import tpu as pltpu
from jax.experimental.pallas import tpu_sc as plsc

_NUM_AGENTS = 4096
_FEAT = 3
_TOTAL = _NUM_AGENTS * _FEAT

_NS = plsc.get_sparse_core_info().num_subcores
_CHUNK = _TOTAL // _NS
assert _CHUNK * _NS == _TOTAL and _CHUNK % 8 == 0


def _body(table_hbm, out_hbm):
    sid = lax.axis_index("s")
    base = sid * _CHUNK
    pltpu.sync_copy(
        table_hbm.at[pl.ds(base, _CHUNK)], out_hbm.at[pl.ds(base, _CHUNK)]
    )


_sc = functools.partial(
    pl.kernel,
    out_type=jax.ShapeDtypeStruct((_TOTAL,), jnp.float32),
    mesh=plsc.VectorSubcoreMesh(
        core_axis_name="c", subcore_axis_name="s", num_cores=1
    ),
)(_body)


def kernel(pos_phi, num_agents):
    flat = jnp.reshape(pos_phi, (-1,))
    out = _sc(flat)
    return jnp.reshape(out, (_NUM_AGENTS, _FEAT))


# speedup vs baseline: 1.0588x; 1.0588x over previous
import functools

import jax
import jax.numpy as jnp
from jax import lax
from jax.experimental import pallas as pl
from jax.experimental.pallas import tpu as pltpu
from jax.experimental.pallas import tpu_sc as plsc

_NUM_AGENTS = 4096
_FEAT = 3
_TOTAL = _NUM_AGENTS * _FEAT

_NS = plsc.get_sparse_core_info().num_subcores
_CHUNK = _TOTAL // _NS
assert _CHUNK * _NS == _TOTAL and _CHUNK % 8 == 0


def _body(table_hbm, out_hbm, buf):
    sid = lax.axis_index("s")
    base = sid * _CHUNK
    pltpu.sync_copy(table_hbm.at[pl.ds(base, _CHUNK)], buf)
    pltpu.sync_copy(buf, out_hbm.at[pl.ds(base, _CHUNK)])


_sc = functools.partial(
    pl.kernel,
    out_type=jax.ShapeDtypeStruct((_TOTAL,), jnp.float32),
    mesh=plsc.VectorSubcoreMesh(
        core_axis_name="c", subcore_axis_name="s", num_cores=1
    ),
    scratch_types=[pltpu.VMEM((_CHUNK,), jnp.float32)],
)(_body)


def kernel(pos_phi, num_agents):
    flat = jnp.reshape(pos_phi, (-1,))
    out = _sc(flat)
    return jnp.reshape(out, (_NUM_AGENTS, _FEAT))
